# baseline (speedup 1.0000x reference)
import jax
import jax.numpy as jnp
from jax.experimental import pallas as pl
from jax.experimental.pallas import tpu as pltpu

_RB = 512
_BB = 4


def _body(tissue_ref, x_ref, pm_ref, reg_ref, out_ref, mask_ref, carry_ref):
    b = pl.program_id(0)
    j = pl.program_id(1)
    n_reg = reg_ref.shape[0]

    @pl.when(j == 0)
    def _first_block():
        for bb in range(_BB):
            t = tissue_ref[b * _BB + bb, 0]
            row_ids = jax.lax.broadcasted_iota(jnp.int32, (n_reg, 1), 0)
            onehot = (row_ids == t).astype(out_ref.dtype)
            out_ref[bb, 0:1, :] = jnp.sum(reg_ref[...] * onehot, axis=0,
                                          keepdims=True)
        mask_ref[:, :, 0:1] = jnp.zeros((_BB, 1, 1), jnp.int32)
        mask_ref[:, :, 1:] = pm_ref[...]

    @pl.when(j > 0)
    def _later_blocks():
        out_ref[:, 0:1, :] = carry_ref[...]

    out_ref[:, 1:, :] = x_ref[:, : _RB - 1, :]
    carry_ref[...] = x_ref[:, _RB - 1 : _RB, :]


def kernel(x, tissue_vector, padding_mask, registry_tokens):
    b_sz, s_sz, d = x.shape
    nj = s_sz // _RB
    pm_i32 = padding_mask.astype(jnp.int32).reshape(b_sz, 1, s_sz)
    out, mask_i32 = pl.pallas_call(
        _body,
        grid=(b_sz // _BB, nj + 1),
        out_shape=[
            jax.ShapeDtypeStruct((b_sz, s_sz + 1, d), x.dtype),
            jax.ShapeDtypeStruct((b_sz, 1, s_sz + 1), jnp.int32),
        ],
        in_specs=[
            pl.BlockSpec(memory_space=pltpu.MemorySpace.SMEM),
            pl.BlockSpec((_BB, _RB, d),
                         lambda b, j: (b, jnp.minimum(j, nj - 1), 0)),
            pl.BlockSpec((_BB, 1, s_sz), lambda b, j: (b, 0, 0)),
            pl.BlockSpec(registry_tokens.shape, lambda b, j: (0, 0)),
        ],
        out_specs=[
            pl.BlockSpec((_BB, _RB, d), lambda b, j: (b, j, 0)),
            pl.BlockSpec((_BB, 1, s_sz + 1), lambda b, j: (b, 0, 0)),
        ],
        scratch_shapes=[pltpu.VMEM((_BB, 1, d), x.dtype)],
    )(tissue_vector, x, pm_i32, registry_tokens)
    return out, mask_i32.reshape(b_sz, s_sz + 1).astype(padding_mask.dtype)


# speedup vs baseline: 1.0007x; 1.0007x over previous
import jax
import jax.numpy as jnp
from jax.experimental import pallas as pl
from jax.experimental.pallas import tpu as pltpu

_CH = 512
_NB = 4


def _in_copy(k, x_ref, inbuf, in_sems, n_chunks):
    b = k // n_chunks
    c = k % n_chunks
    s = jax.lax.rem(k, _NB)
    return pltpu.make_async_copy(
        x_ref.at[b, pl.ds(c * _CH, _CH), :], inbuf.at[s], in_sems.at[s])


def _out_copy(k, out_ref, outbuf, out_sems, n_chunks):
    b = k // n_chunks
    c = k % n_chunks
    s = jax.lax.rem(k, _NB)
    return pltpu.make_async_copy(
        outbuf.at[s], out_ref.at[b, pl.ds(c * _CH, _CH), :], out_sems.at[s])


def _body(tissue_ref, x_ref, pm_ref, reg_ref, out_ref, mask_ref,
          inbuf, outbuf, carry, tails, in_sems, out_sems, tail_sem):
    b_sz, s_sz, d = x_ref.shape
    n_chunks = s_sz // _CH
    n_total = b_sz * n_chunks
    n_reg = reg_ref.shape[0]

    mask_ref[:, :, 0:1] = jnp.zeros((b_sz, 1, 1), jnp.int32)
    mask_ref[:, :, 1:] = pm_ref[...]

    for k in range(_NB):
        _in_copy(k, x_ref, inbuf, in_sems, n_chunks).start()

    def step(k, _):
        b = k // n_chunks
        c = k % n_chunks
        s = jax.lax.rem(k, _NB)
        _in_copy(k, x_ref, inbuf, in_sems, n_chunks).wait()

        @pl.when(k >= _NB)
        def _():
            _out_copy(k - _NB, out_ref, outbuf, out_sems, n_chunks).wait()

        @pl.when(c == 0)
        def _():
            t = tissue_ref[b, 0]
            row_ids = jax.lax.broadcasted_iota(jnp.int32, (n_reg, 1), 0)
            onehot = (row_ids == t).astype(jnp.float32)
            outbuf[s, 0:1, :] = jnp.sum(reg_ref[...] * onehot, axis=0,
                                        keepdims=True)

        @pl.when(c > 0)
        def _():
            outbuf[s, 0:1, :] = carry[...]

        outbuf[s, 1:, :] = inbuf[s, : _CH - 1, :]
        carry[...] = inbuf[s, _CH - 1 : _CH, :]

        @pl.when(c == n_chunks - 1)
        def _():
            tails[b, :, :] = inbuf[s, _CH - 1 : _CH, :]

        _out_copy(k, out_ref, outbuf, out_sems, n_chunks).start()

        @pl.when(k + _NB < n_total)
        def _():
            _in_copy(k + _NB, x_ref, inbuf, in_sems, n_chunks).start()

        return 0

    jax.lax.fori_loop(0, n_total, step, 0)

    for k in range(n_total - _NB, n_total):
        _out_copy(k, out_ref, outbuf, out_sems, n_chunks).wait()

    tail = pltpu.make_async_copy(
        tails, out_ref.at[:, pl.ds(s_sz, 1), :], tail_sem)
    tail.start()
    tail.wait()


def kernel(x, tissue_vector, padding_mask, registry_tokens):
    b_sz, s_sz, d = x.shape
    pm_i32 = padding_mask.astype(jnp.int32).reshape(b_sz, 1, s_sz)
    out, mask_i32 = pl.pallas_call(
        _body,
        out_shape=[
            jax.ShapeDtypeStruct((b_sz, s_sz + 1, d), x.dtype),
            jax.ShapeDtypeStruct((b_sz, 1, s_sz + 1), jnp.int32),
        ],
        in_specs=[
            pl.BlockSpec(memory_space=pltpu.MemorySpace.SMEM),
            pl.BlockSpec(memory_space=pltpu.MemorySpace.HBM),
            pl.BlockSpec(memory_space=pltpu.MemorySpace.VMEM),
            pl.BlockSpec(memory_space=pltpu.MemorySpace.VMEM),
        ],
        out_specs=[
            pl.BlockSpec(memory_space=pltpu.MemorySpace.HBM),
            pl.BlockSpec(memory_space=pltpu.MemorySpace.VMEM),
        ],
        scratch_shapes=[
            pltpu.VMEM((_NB, _CH, d), x.dtype),
            pltpu.VMEM((_NB, _CH, d), x.dtype),
            pltpu.VMEM((1, d), x.dtype),
            pltpu.VMEM((b_sz, 1, d), x.dtype),
            pltpu.SemaphoreType.DMA((_NB,)),
            pltpu.SemaphoreType.DMA((_NB,)),
            pltpu.SemaphoreType.DMA,
        ],
    )(tissue_vector, x, pm_i32, registry_tokens)
    return out, mask_i32.reshape(b_sz, s_sz + 1).astype(padding_mask.dtype)
